# baseline (speedup 1.0000x reference)
import functools

import jax
import jax.numpy as jnp
from jax import lax
from jax.experimental import pallas as pl
from jax.experimental.pallas import tpu as pltpu
from jax.experimental.pallas import tpu_sc as plsc

N = 10000
D = 128
QD = D // 4
NC = 2
NS = 16
NW = NC * NS
L = 16

C = 128
NB = 2
N_SP = 10240
ZR = N_SP // NS


def _sc_aggregate(xq, src_r, dst_r, cpt):
    mesh = plsc.VectorSubcoreMesh(core_axis_name="c", subcore_axis_name="s")

    @functools.partial(
        pl.kernel,
        out_type=(
            jax.ShapeDtypeStruct((4, N_SP, QD), jnp.float32),
            jax.ShapeDtypeStruct((NW * N,), jnp.float32),
        ),
        mesh=mesh,
        compiler_params=pltpu.CompilerParams(
            needs_layout_passes=False, use_tc_tiling_on_sc=False
        ),
        scratch_types=(
            pltpu.VMEM((cpt + NB, C), jnp.int32),
            pltpu.VMEM((cpt, C), jnp.int32),
            tuple(pltpu.VMEM((C, QD), jnp.float32) for _ in range(NB)),
            pltpu.VMEM((N_SP,), jnp.float32),
            pltpu.VMEM_SHARED((N_SP, QD), jnp.float32),
            pltpu.VMEM_SHARED((N_SP, QD), jnp.float32),
            tuple(pltpu.SemaphoreType.DMA for _ in range(NB)),
            pltpu.SemaphoreType.DMA,
            pltpu.SemaphoreType.DMA,
        ),
    )
    def run(x_hbm, src_hbm, dst_hbm, agg_out, cnt_out,
            src_v, dst_v, bufs, cnt_v, agg_sp, x_sp, gsems, ssem, xsem):
        rows0 = bufs[0]
        c = lax.axis_index("c")
        s = lax.axis_index("s")
        wid = c * NS + s
        zbase = s * ZR

        zvec = jnp.zeros((L,), jnp.float32)

        def zrow(i, carry):
            for k in range(QD // L):
                rows0[i, pl.ds(k * L, L)] = zvec
            return carry

        def zcnt(i, carry):
            cnt_v[pl.ds(i * L, L)] = zvec
            return carry

        lax.fori_loop(0, N_SP // L, zcnt, 0)

        pltpu.sync_copy(src_hbm.at[wid], src_v)
        pltpu.sync_copy(dst_hbm.at[wid], dst_v)

        def gather_start(chunk, buf, sem):
            pltpu.async_copy(x_sp.at[src_v.at[chunk]], buf, sem)

        def gather_wait(buf, sem):
            pltpu.make_async_copy(x_sp.at[src_v.at[0]], buf, sem).wait()

        def scatter_start(chunk, buf):
            pltpu.async_copy(buf, agg_sp.at[dst_v.at[chunk]], ssem, add=True)

        def scatter_wait(buf):
            pltpu.make_async_copy(buf, agg_sp.at[dst_v.at[0]], ssem).wait()

        ones = jnp.full((L,), 1.0, jnp.float32)

        def count(chunk):
            for k in range(C // L):
                idx = dst_v[chunk, pl.ds(k * L, L)]
                plsc.addupdate_scatter(cnt_v, [idx], ones)

        for p in range(2):
            q = 2 * p + c

            pltpu.async_copy(x_hbm.at[q, pl.ds(zbase, ZR)],
                             x_sp.at[pl.ds(zbase, ZR)], xsem)

            lax.fori_loop(0, C, zrow, 0)
            for k in range(ZR // C):
                pltpu.sync_copy(rows0.at[pl.ds(0, C)],
                                agg_sp.at[pl.ds(zbase + k * C, C)])
            pltpu.make_async_copy(x_hbm.at[0, pl.ds(0, ZR)],
                                  x_sp.at[pl.ds(zbase, ZR)], xsem).wait()
            plsc.subcore_barrier()

            for b in range(NB):
                gather_start(b, bufs[b], gsems[b])

            def mbody(i, carry):
                base = NB * i
                for b in range(NB):
                    gather_wait(bufs[b], gsems[b])
                    scatter_start(base + b, bufs[b])
                if p == 0:
                    count(base)
                scatter_wait(bufs[0])
                gather_start(base + NB, bufs[0], gsems[0])
                if p == 0:
                    count(base + 1)
                scatter_wait(bufs[1])
                gather_start(base + NB + 1, bufs[1], gsems[1])
                return carry

            lax.fori_loop(0, cpt // NB, mbody, 0)
            for b in range(NB):
                gather_wait(bufs[b], gsems[b])
            plsc.subcore_barrier()

            pltpu.sync_copy(agg_sp.at[pl.ds(zbase, ZR)],
                            agg_out.at[q, pl.ds(zbase, ZR)])
            if p == 0:
                plsc.subcore_barrier()

        pltpu.sync_copy(cnt_v.at[pl.ds(0, N)],
                        cnt_out.at[pl.ds(pl.multiple_of(wid * N, 8), N)])

    return run(xq, src_r, dst_r)


def _tc_body(p_ref, cnt_ref, x_ref, wl_ref, wr_ref, b_ref, o_ref):
    agg = jnp.concatenate([p_ref[0], p_ref[1], p_ref[2], p_ref[3]], axis=-1)
    cnt = 0.5 * jnp.sum(cnt_ref[...], axis=1, keepdims=True)
    mean = agg / jnp.clip(cnt, 1.0, None)
    o_ref[...] = (
        jnp.dot(mean, wl_ref[...], preferred_element_type=jnp.float32)
        + jnp.dot(x_ref[...], wr_ref[...], preferred_element_type=jnp.float32)
        + b_ref[...]
    )


def _tc_finalize(agg, cnt_t, x, W_l, W_r, b2):
    br = 400
    return pl.pallas_call(
        _tc_body,
        grid=(N // br,),
        in_specs=[
            pl.BlockSpec((4, br, QD), lambda i: (0, i, 0)),
            pl.BlockSpec((br, NW), lambda i: (i, 0)),
            pl.BlockSpec((br, D), lambda i: (i, 0)),
            pl.BlockSpec((D, D), lambda i: (0, 0)),
            pl.BlockSpec((D, D), lambda i: (0, 0)),
            pl.BlockSpec((1, D), lambda i: (0, 0)),
        ],
        out_specs=pl.BlockSpec((br, D), lambda i: (i, 0)),
        out_shape=jax.ShapeDtypeStruct((N, D), jnp.float32),
    )(agg, cnt_t, x, W_l, W_r, b2)


def kernel(x, edge_index, W_l, W_r, b):
    e = edge_index.shape[1]
    src = edge_index[0].astype(jnp.int32)
    dst = edge_index[1].astype(jnp.int32)

    pad = jnp.zeros((N_SP - N, QD), jnp.float32)
    xq = jnp.stack([jnp.concatenate([x[:, i * QD:(i + 1) * QD], pad], axis=0)
                    for i in range(4)])

    cpt = -(-e // (NS * C * NB)) * NB
    e_pad = NS * cpt * C
    src_p = jnp.concatenate([src, jnp.zeros((e_pad - e,), jnp.int32)])
    dst_p = jnp.concatenate([dst, jnp.full((e_pad - e,), N, jnp.int32)])
    src16 = src_p.reshape(NS, cpt, C)
    dst16 = dst_p.reshape(NS, cpt, C)
    src_r = jnp.concatenate([src16, src16], axis=0)
    src_r = jnp.concatenate([src_r, jnp.zeros((NW, NB, C), jnp.int32)], axis=1)
    dst_r = jnp.concatenate([dst16, dst16], axis=0)

    agg, cnt = _sc_aggregate(xq, src_r, dst_r, cpt)
    cnt_t = cnt.reshape(NW, N).T
    return _tc_finalize(agg, cnt_t, x, W_l, W_r, b.reshape(1, D))


# speedup vs baseline: 1.2418x; 1.2418x over previous
import functools

import jax
import jax.numpy as jnp
from jax import lax
from jax.experimental import pallas as pl
from jax.experimental.pallas import tpu as pltpu
from jax.experimental.pallas import tpu_sc as plsc

N = 10000
D = 128
QD = D // 4
NC = 2
NS = 16
NW = NC * NS
L = 16

C = 128
NB = 2
N_SP = 10240
ZR = N_SP // NS


def _sc_aggregate(x, src_r, dst_r, cpt):
    mesh = plsc.VectorSubcoreMesh(core_axis_name="c", subcore_axis_name="s")

    @functools.partial(
        pl.kernel,
        out_type=(
            jax.ShapeDtypeStruct((4, N_SP, QD), jnp.float32),
            jax.ShapeDtypeStruct((NW * N,), jnp.float32),
        ),
        mesh=mesh,
        compiler_params=pltpu.CompilerParams(
            needs_layout_passes=False, use_tc_tiling_on_sc=False
        ),
        scratch_types=(
            pltpu.VMEM((cpt + NB, C), jnp.int32),
            pltpu.VMEM((cpt, C), jnp.int32),
            tuple(pltpu.VMEM((C, QD), jnp.float32) for _ in range(NB)),
            pltpu.VMEM((N_SP,), jnp.float32),
            pltpu.VMEM_SHARED((N_SP, QD), jnp.float32),
            pltpu.VMEM_SHARED((N_SP, QD), jnp.float32),
            tuple(pltpu.SemaphoreType.DMA for _ in range(NB)),
            pltpu.SemaphoreType.DMA,
            pltpu.SemaphoreType.DMA,
        ),
    )
    def run(x_hbm, src_hbm, dst_hbm, agg_out, cnt_out,
            src_v, dst_v, bufs, cnt_v, agg_sp, x_sp, gsems, ssem, xsem):
        rows0 = bufs[0]
        c = lax.axis_index("c")
        s = lax.axis_index("s")
        wid = c * NS + s
        zbase = s * ZR

        zvec = jnp.zeros((L,), jnp.float32)

        def zrow(i, carry):
            for k in range(QD // L):
                rows0[i, pl.ds(k * L, L)] = zvec
            return carry

        def zcnt(i, carry):
            cnt_v[pl.ds(i * L, L)] = zvec
            return carry

        lax.fori_loop(0, N_SP // L, zcnt, 0)

        pltpu.sync_copy(src_hbm.at[s], src_v)
        pltpu.sync_copy(dst_hbm.at[s], dst_v)

        def gather_start(chunk, buf, sem):
            pltpu.async_copy(x_sp.at[src_v.at[chunk]], buf, sem)

        def gather_wait(buf, sem):
            pltpu.make_async_copy(x_sp.at[src_v.at[0]], buf, sem).wait()

        def scatter_start(chunk, buf):
            pltpu.async_copy(buf, agg_sp.at[dst_v.at[chunk]], ssem, add=True)

        def scatter_wait(buf):
            pltpu.make_async_copy(buf, agg_sp.at[dst_v.at[0]], ssem).wait()

        ones = jnp.full((L,), 1.0, jnp.float32)

        def count(chunk):
            for k in range(C // L):
                idx = dst_v[chunk, pl.ds(k * L, L)]
                plsc.addupdate_scatter(cnt_v, [idx], ones)

        SR = 632
        SRL = N - (NS - 1) * SR
        sr0 = s * SR

        def stage_start(q):
            @pl.when(s < NS - 1)
            def _():
                pltpu.async_copy(x_hbm.at[pl.ds(sr0, SR), pl.ds(q * QD, QD)],
                                 x_sp.at[pl.ds(sr0, SR)], xsem)

            @pl.when(s == NS - 1)
            def _():
                pltpu.async_copy(x_hbm.at[pl.ds(sr0, SRL), pl.ds(q * QD, QD)],
                                 x_sp.at[pl.ds(sr0, SRL)], xsem)

        def stage_wait():
            @pl.when(s < NS - 1)
            def _():
                pltpu.make_async_copy(x_hbm.at[pl.ds(0, SR), pl.ds(0, QD)],
                                      x_sp.at[pl.ds(0, SR)], xsem).wait()

            @pl.when(s == NS - 1)
            def _():
                pltpu.make_async_copy(x_hbm.at[pl.ds(0, SRL), pl.ds(0, QD)],
                                      x_sp.at[pl.ds(0, SRL)], xsem).wait()

        for p in range(2):
            q = 2 * p + c

            stage_start(q)

            lax.fori_loop(0, C, zrow, 0)
            for k in range(ZR // C):
                pltpu.sync_copy(rows0.at[pl.ds(0, C)],
                                agg_sp.at[pl.ds(zbase + k * C, C)])
            stage_wait()
            plsc.subcore_barrier()

            for b in range(NB):
                gather_start(b, bufs[b], gsems[b])

            def mbody(i, carry):
                base = NB * i
                for b in range(NB):
                    gather_wait(bufs[b], gsems[b])
                    scatter_start(base + b, bufs[b])
                if p == 0:
                    count(base)
                scatter_wait(bufs[0])
                gather_start(base + NB, bufs[0], gsems[0])
                if p == 0:
                    count(base + 1)
                scatter_wait(bufs[1])
                gather_start(base + NB + 1, bufs[1], gsems[1])
                return carry

            lax.fori_loop(0, cpt // NB, mbody, 0)
            for b in range(NB):
                gather_wait(bufs[b], gsems[b])
            plsc.subcore_barrier()

            pltpu.sync_copy(agg_sp.at[pl.ds(zbase, ZR)],
                            agg_out.at[q, pl.ds(zbase, ZR)])
            if p == 0:
                plsc.subcore_barrier()

        pltpu.sync_copy(cnt_v.at[pl.ds(0, N)],
                        cnt_out.at[pl.ds(pl.multiple_of(wid * N, 8), N)])

    return run(x, src_r, dst_r)


def _tc_body(p_ref, cnt_ref, x_ref, wl_ref, wr_ref, b_ref, o_ref):
    agg = jnp.concatenate([p_ref[0], p_ref[1], p_ref[2], p_ref[3]], axis=-1)
    cnt = 0.5 * jnp.sum(cnt_ref[...], axis=1, keepdims=True)
    mean = agg / jnp.clip(cnt, 1.0, None)
    o_ref[...] = (
        jnp.dot(mean, wl_ref[...], preferred_element_type=jnp.float32)
        + jnp.dot(x_ref[...], wr_ref[...], preferred_element_type=jnp.float32)
        + b_ref[...]
    )


def _tc_finalize(agg, cnt_t, x, W_l, W_r, b2):
    br = 400
    return pl.pallas_call(
        _tc_body,
        grid=(N // br,),
        in_specs=[
            pl.BlockSpec((4, br, QD), lambda i: (0, i, 0)),
            pl.BlockSpec((br, NW), lambda i: (i, 0)),
            pl.BlockSpec((br, D), lambda i: (i, 0)),
            pl.BlockSpec((D, D), lambda i: (0, 0)),
            pl.BlockSpec((D, D), lambda i: (0, 0)),
            pl.BlockSpec((1, D), lambda i: (0, 0)),
        ],
        out_specs=pl.BlockSpec((br, D), lambda i: (i, 0)),
        out_shape=jax.ShapeDtypeStruct((N, D), jnp.float32),
    )(agg, cnt_t, x, W_l, W_r, b2)


def kernel(x, edge_index, W_l, W_r, b):
    e = edge_index.shape[1]
    src = edge_index[0].astype(jnp.int32)
    dst = edge_index[1].astype(jnp.int32)

    cpt = -(-e // (NS * C * NB)) * NB
    e_pad = NS * cpt * C
    src_p = jnp.concatenate([src, jnp.zeros((e_pad - e,), jnp.int32)])
    dst_p = jnp.concatenate([dst, jnp.full((e_pad - e,), N, jnp.int32)])
    src16 = src_p.reshape(NS, cpt, C)
    dst16 = dst_p.reshape(NS, cpt, C)
    src16 = jnp.concatenate([src16, jnp.zeros((NS, NB, C), jnp.int32)], axis=1)

    agg, cnt = _sc_aggregate(x, src16, dst16, cpt)
    cnt_t = cnt.reshape(NW, N).T
    return _tc_finalize(agg, cnt_t, x, W_l, W_r, b.reshape(1, D))


# speedup vs baseline: 1.2687x; 1.0217x over previous
import functools

import jax
import jax.numpy as jnp
from jax import lax
from jax.experimental import pallas as pl
from jax.experimental.pallas import tpu as pltpu
from jax.experimental.pallas import tpu_sc as plsc

N = 10000
D = 128
QD = D // 4
NC = 2
NS = 16
NW = NC * NS
L = 16

C = 128
NB = 2
N_SP = 10240
ZR = N_SP // NS


def _sc_aggregate(x, src_r, dst_r, cpt):
    mesh = plsc.VectorSubcoreMesh(core_axis_name="c", subcore_axis_name="s")

    @functools.partial(
        pl.kernel,
        out_type=(
            jax.ShapeDtypeStruct((4, N_SP, QD), jnp.float32),
            jax.ShapeDtypeStruct((NC, N_SP), jnp.float32),
        ),
        mesh=mesh,
        compiler_params=pltpu.CompilerParams(
            needs_layout_passes=False, use_tc_tiling_on_sc=False
        ),
        scratch_types=(
            pltpu.VMEM((cpt + NB, C), jnp.int32),
            pltpu.VMEM((cpt, C), jnp.int32),
            tuple(pltpu.VMEM((C, QD), jnp.float32) for _ in range(NB)),
            pltpu.VMEM((N_SP,), jnp.float32),
            pltpu.VMEM((NS, ZR), jnp.float32),
            pltpu.VMEM_SHARED((N_SP, QD), jnp.float32),
            pltpu.VMEM_SHARED((N_SP, QD), jnp.float32),
            pltpu.VMEM_SHARED((NS, N_SP), jnp.float32),
            tuple(pltpu.SemaphoreType.DMA for _ in range(NB)),
            pltpu.SemaphoreType.DMA,
            pltpu.SemaphoreType.DMA,
        ),
    )
    def run(x_hbm, src_hbm, dst_hbm, agg_out, cnt_out,
            src_v, dst_v, bufs, cnt_v, red_v, agg_sp, x_sp, cnt_sp,
            gsems, ssem, xsem):
        rows0 = bufs[0]
        c = lax.axis_index("c")
        s = lax.axis_index("s")
        zbase = s * ZR

        zvec = jnp.zeros((L,), jnp.float32)

        def zrow(i, carry):
            for k in range(QD // L):
                rows0[i, pl.ds(k * L, L)] = zvec
            return carry

        def zcnt(i, carry):
            cnt_v[pl.ds(i * L, L)] = zvec
            return carry

        lax.fori_loop(0, N_SP // L, zcnt, 0)

        pltpu.sync_copy(src_hbm.at[s], src_v)
        pltpu.sync_copy(dst_hbm.at[s], dst_v)

        def gather_start(chunk, buf, sem):
            pltpu.async_copy(x_sp.at[src_v.at[chunk]], buf, sem)

        def gather_wait(buf, sem):
            pltpu.make_async_copy(x_sp.at[src_v.at[0]], buf, sem).wait()

        def scatter_start(chunk, buf):
            pltpu.async_copy(buf, agg_sp.at[dst_v.at[chunk]], ssem, add=True)

        def scatter_wait(buf):
            pltpu.make_async_copy(buf, agg_sp.at[dst_v.at[0]], ssem).wait()

        ones = jnp.full((L,), 1.0, jnp.float32)

        def count(chunk):
            for k in range(C // L):
                idx = dst_v[chunk, pl.ds(k * L, L)]
                plsc.addupdate_scatter(cnt_v, [idx], ones)

        SR = 632
        SRL = N - (NS - 1) * SR
        sr0 = s * SR

        def stage_start(q):
            @pl.when(s < NS - 1)
            def _():
                pltpu.async_copy(x_hbm.at[pl.ds(sr0, SR), pl.ds(q * QD, QD)],
                                 x_sp.at[pl.ds(sr0, SR)], xsem)

            @pl.when(s == NS - 1)
            def _():
                pltpu.async_copy(x_hbm.at[pl.ds(sr0, SRL), pl.ds(q * QD, QD)],
                                 x_sp.at[pl.ds(sr0, SRL)], xsem)

        def stage_wait():
            @pl.when(s < NS - 1)
            def _():
                pltpu.make_async_copy(x_hbm.at[pl.ds(0, SR), pl.ds(0, QD)],
                                      x_sp.at[pl.ds(0, SR)], xsem).wait()

            @pl.when(s == NS - 1)
            def _():
                pltpu.make_async_copy(x_hbm.at[pl.ds(0, SRL), pl.ds(0, QD)],
                                      x_sp.at[pl.ds(0, SRL)], xsem).wait()

        for p in range(2):
            q = 2 * p + c

            stage_start(q)

            lax.fori_loop(0, C, zrow, 0)
            for k in range(ZR // C):
                pltpu.sync_copy(rows0.at[pl.ds(0, C)],
                                agg_sp.at[pl.ds(zbase + k * C, C)])
            stage_wait()
            plsc.subcore_barrier()

            for b in range(NB):
                gather_start(b, bufs[b], gsems[b])

            def mbody(i, carry):
                base = NB * i
                for b in range(NB):
                    gather_wait(bufs[b], gsems[b])
                    scatter_start(base + b, bufs[b])
                if p == 0:
                    count(base)
                scatter_wait(bufs[0])
                gather_start(base + NB, bufs[0], gsems[0])
                if p == 0:
                    count(base + 1)
                scatter_wait(bufs[1])
                gather_start(base + NB + 1, bufs[1], gsems[1])
                return carry

            lax.fori_loop(0, cpt // NB, mbody, 0)
            for b in range(NB):
                gather_wait(bufs[b], gsems[b])
            plsc.subcore_barrier()

            pltpu.sync_copy(agg_sp.at[pl.ds(zbase, ZR)],
                            agg_out.at[q, pl.ds(zbase, ZR)])
            if p == 0:
                pltpu.sync_copy(cnt_v, cnt_sp.at[s])
                plsc.subcore_barrier()

        pltpu.sync_copy(cnt_sp.at[:, pl.ds(zbase, ZR)], red_v)

        def rbody(j, carry):
            acc = red_v[0, pl.ds(j * L, L)]
            for t in range(1, NS):
                acc = acc + red_v[t, pl.ds(j * L, L)]
            cnt_v[pl.ds(j * L, L)] = acc
            return carry

        lax.fori_loop(0, ZR // L, rbody, 0)
        pltpu.sync_copy(cnt_v.at[pl.ds(0, ZR)], cnt_out.at[c, pl.ds(zbase, ZR)])

    return run(x, src_r, dst_r)


def _tc_body(p_ref, cnt_ref, x_ref, wl_ref, wr_ref, b_ref, o_ref):
    agg = jnp.concatenate([p_ref[0], p_ref[1], p_ref[2], p_ref[3]], axis=-1)
    cnt = 0.5 * (cnt_ref[:, 0:1] + cnt_ref[:, 1:2])
    mean = agg / jnp.clip(cnt, 1.0, None)
    o_ref[...] = (
        jnp.dot(mean, wl_ref[...], preferred_element_type=jnp.float32)
        + jnp.dot(x_ref[...], wr_ref[...], preferred_element_type=jnp.float32)
        + b_ref[...]
    )


def _tc_finalize(agg, cnt_t, x, W_l, W_r, b2):
    br = 400
    return pl.pallas_call(
        _tc_body,
        grid=(N // br,),
        in_specs=[
            pl.BlockSpec((4, br, QD), lambda i: (0, i, 0)),
            pl.BlockSpec((br, NC), lambda i: (i, 0)),
            pl.BlockSpec((br, D), lambda i: (i, 0)),
            pl.BlockSpec((D, D), lambda i: (0, 0)),
            pl.BlockSpec((D, D), lambda i: (0, 0)),
            pl.BlockSpec((1, D), lambda i: (0, 0)),
        ],
        out_specs=pl.BlockSpec((br, D), lambda i: (i, 0)),
        out_shape=jax.ShapeDtypeStruct((N, D), jnp.float32),
    )(agg, cnt_t, x, W_l, W_r, b2)


def kernel(x, edge_index, W_l, W_r, b):
    e = edge_index.shape[1]
    src = edge_index[0].astype(jnp.int32)
    dst = edge_index[1].astype(jnp.int32)

    cpt = -(-e // (NS * C * NB)) * NB
    e_pad = NS * cpt * C
    src_p = jnp.concatenate([src, jnp.zeros((e_pad - e,), jnp.int32)])
    dst_p = jnp.concatenate([dst, jnp.full((e_pad - e,), N, jnp.int32)])
    src16 = src_p.reshape(NS, cpt, C)
    dst16 = dst_p.reshape(NS, cpt, C)
    src16 = jnp.concatenate([src16, jnp.zeros((NS, NB, C), jnp.int32)], axis=1)

    agg, cnt = _sc_aggregate(x, src16, dst16, cpt)
    return _tc_finalize(agg, cnt.T, x, W_l, W_r, b.reshape(1, D))
